# baseline (speedup 1.0000x reference)
import functools

import jax
import jax.numpy as jnp
from jax.experimental import pallas as pl

MAX_RADIUS = 10.0


def _dist_kernel(c_ref, p_ref, o_ref):
    c = c_ref[0]
    p = p_ref[0]
    a_sq = jnp.sum(c * c, axis=-1, keepdims=True)
    b_sq = jnp.sum(p * p, axis=-1)[None, :]
    cb = c.astype(jnp.bfloat16).astype(jnp.float32)
    pb = p.astype(jnp.bfloat16).astype(jnp.float32)
    inner = (cb[:, 0:1] * pb[:, 0][None, :]
             + cb[:, 1:2] * pb[:, 1][None, :]
             + cb[:, 2:3] * pb[:, 2][None, :])
    d2 = jnp.clip(a_sq + b_sq - 2.0 * inner, 0.0, None)
    o_ref[0] = jnp.sqrt(d2 + 1e-08)


def _pairwise_dist(centers, points):
    B, M, _ = centers.shape
    N = points.shape[1]
    BM = 256
    return pl.pallas_call(
        _dist_kernel,
        grid=(B, M // BM),
        in_specs=[
            pl.BlockSpec((1, BM, 3), lambda b, i: (b, i, 0)),
            pl.BlockSpec((1, N, 3), lambda b, i: (b, 0, 0)),
        ],
        out_specs=pl.BlockSpec((1, BM, N), lambda b, i: (b, i, 0)),
        out_shape=jax.ShapeDtypeStruct((B, M, N), jnp.float32),
    )(centers, points)


def kernel(points, feats, center_idx, W1, b1, W2, b2, W3, b3):
    Bn, Nn, _ = points.shape
    Mn = center_idx.shape[1]
    K = 32
    centers = jnp.take_along_axis(
        points, jnp.broadcast_to(center_idx[:, :, None], (Bn, Mn, 3)), axis=1)
    dist = _pairwise_dist(centers, points)
    mask = dist <= MAX_RADIUS
    masked_dist = jnp.where(mask, dist, jnp.inf)
    _, knn_idx = jax.lax.top_k(-masked_dist, K)
    knn_idx = jnp.clip(knn_idx, None, Nn - 1)
    neigh_feats = feats[jnp.arange(Bn)[:, None, None], knn_idx]
    centers_feats = jnp.take_along_axis(
        feats, jnp.broadcast_to(center_idx[:, :, None], (Bn, Mn, feats.shape[-1])), axis=1)
    C = feats.shape[-1]
    centers_feats_exp = jnp.broadcast_to(centers_feats[:, :, None, :], (Bn, Mn, K, C))
    combo = jnp.concatenate([centers_feats_exp, neigh_feats], axis=-1)
    combo2 = combo.reshape(Bn * Mn * K, 2 * C)
    h = jax.nn.relu(combo2 @ W1.T + b1)
    channel_w = jax.nn.sigmoid(h @ W2.T + b2)
    channel_w = channel_w.reshape(Bn, Mn, K, C)
    fused_nei = jnp.mean(neigh_feats * channel_w, axis=2)
    fused = fused_nei + centers_feats
    out = jax.nn.relu(fused @ W3.T + b3)
    return (out, knn_idx)


# speedup vs baseline: 1.6440x; 1.6440x over previous
import jax
import jax.numpy as jnp
from jax.experimental import pallas as pl

MAXR = 10.0
KNN = 32
NLANE = 128
LLIST = 8


def _dist_topk_kernel(c_ref, pt_ref, o_ref):
    c = c_ref[0]
    pt = pt_ref[0]
    MT = c.shape[0]
    N = pt.shape[1]
    f32 = jnp.float32

    c0 = c[:, 0:1]
    c1 = c[:, 1:2]
    c2 = c[:, 2:3]
    p0 = pt[0:1, :]
    p1 = pt[1:2, :]
    p2 = pt[2:3, :]

    a_sq = c0 * c0 + c1 * c1 + c2 * c2
    b_sq = p0 * p0 + p1 * p1 + p2 * p2

    cb0 = c0.astype(jnp.bfloat16).astype(f32)
    cb1 = c1.astype(jnp.bfloat16).astype(f32)
    cb2 = c2.astype(jnp.bfloat16).astype(f32)
    pb0 = p0.astype(jnp.bfloat16).astype(f32)
    pb1 = p1.astype(jnp.bfloat16).astype(f32)
    pb2 = p2.astype(jnp.bfloat16).astype(f32)

    q0 = cb0 * pb0
    q1 = cb1 * pb1
    q2 = cb2 * pb2
    s = q0 + q1
    bv = s - q0
    av = s - bv
    e = (q0 - av) + (q1 - bv)
    t = s + q2
    bv2 = t - s
    av2 = t - bv2
    f = (s - av2) + (q2 - bv2)
    inner = t + (e + f)

    d2 = jnp.maximum((a_sq + b_sq) - 2.0 * inner, 0.0)
    dist = jnp.sqrt(d2 + 1e-08)

    nchunk = N // NLANE
    INF = jnp.float32(jnp.inf)

    svals = [jnp.full((MT, NLANE), INF, f32) for _ in range(LLIST)]
    schnk = [jnp.zeros((MT, NLANE), jnp.int32) for _ in range(LLIST)]

    for j in range(nchunk):
        cv = dist[:, j * NLANE:(j + 1) * NLANE]
        ci = jnp.full((MT, NLANE), j, jnp.int32)
        for sl in range(LLIST):
            lt = cv < svals[sl]
            nsv = jnp.where(lt, cv, svals[sl])
            nsc = jnp.where(lt, ci, schnk[sl])
            cv = jnp.where(lt, svals[sl], cv)
            ci = jnp.where(lt, schnk[sl], ci)
            svals[sl] = nsv
            schnk[sl] = nsc

    lane = jax.lax.broadcasted_iota(jnp.int32, (MT, NLANE), 1)
    nidx = [schnk[sl] * NLANE + lane for sl in range(LLIST)]

    BIGI = jnp.int32(2**30)
    winners = []
    for _k in range(KNN):
        m = svals[0]
        for sl in range(1, LLIST):
            m = jnp.minimum(m, svals[sl])
        rowmin = jnp.min(m, axis=-1, keepdims=True)
        cand = jnp.full((MT, NLANE), BIGI, jnp.int32)
        for sl in range(LLIST):
            cand = jnp.minimum(cand, jnp.where(svals[sl] == rowmin, nidx[sl], BIGI))
        rowidx = jnp.min(cand, axis=-1, keepdims=True)
        winners.append(rowidx)
        for sl in range(LLIST):
            svals[sl] = jnp.where(nidx[sl] == rowidx, INF, svals[sl])

    o_ref[0] = jnp.concatenate(winners, axis=1)


def _dist_topk(centers, points_t):
    B, M, _ = centers.shape
    N = points_t.shape[2]
    MT = 8
    return pl.pallas_call(
        _dist_topk_kernel,
        grid=(B, M // MT),
        in_specs=[
            pl.BlockSpec((1, MT, 3), lambda b, i: (b, i, 0)),
            pl.BlockSpec((1, 3, N), lambda b, i: (b, 0, 0)),
        ],
        out_specs=pl.BlockSpec((1, MT, KNN), lambda b, i: (b, i, 0)),
        out_shape=jax.ShapeDtypeStruct((B, M, KNN), jnp.int32),
    )(centers, points_t)


def kernel(points, feats, center_idx, W1, b1, W2, b2, W3, b3):
    Bn, Nn, _ = points.shape
    Mn = center_idx.shape[1]
    K = KNN
    C = feats.shape[-1]
    centers = jnp.take_along_axis(
        points, jnp.broadcast_to(center_idx[:, :, None], (Bn, Mn, 3)), axis=1)
    points_t = jnp.transpose(points, (0, 2, 1))
    knn_idx = _dist_topk(centers, points_t)
    neigh_feats = feats[jnp.arange(Bn)[:, None, None], knn_idx]
    centers_feats = jnp.take_along_axis(
        feats, jnp.broadcast_to(center_idx[:, :, None], (Bn, Mn, C)), axis=1)
    centers_feats_exp = jnp.broadcast_to(centers_feats[:, :, None, :], (Bn, Mn, K, C))
    combo = jnp.concatenate([centers_feats_exp, neigh_feats], axis=-1)
    combo2 = combo.reshape(Bn * Mn * K, 2 * C)
    h = jax.nn.relu(combo2 @ W1.T + b1)
    channel_w = jax.nn.sigmoid(h @ W2.T + b2)
    channel_w = channel_w.reshape(Bn, Mn, K, C)
    fused_nei = jnp.mean(neigh_feats * channel_w, axis=2)
    fused = fused_nei + centers_feats
    out = jax.nn.relu(fused @ W3.T + b3)
    return (out, knn_idx)


# speedup vs baseline: 1.7740x; 1.0791x over previous
import jax
import jax.numpy as jnp
from jax.experimental import pallas as pl

MAXR = 10.0
KNN = 32
NLANE = 128
LLIST = 8


def _dist_topk_kernel(c_ref, pt_ref, o_ref):
    c = c_ref[0]
    pt = pt_ref[0]
    MT = c.shape[0]
    N = pt.shape[1]
    f32 = jnp.float32

    c0 = c[:, 0:1]
    c1 = c[:, 1:2]
    c2 = c[:, 2:3]
    p0 = pt[0:1, :]
    p1 = pt[1:2, :]
    p2 = pt[2:3, :]

    a_sq = c0 * c0 + c1 * c1 + c2 * c2
    b_sq = p0 * p0 + p1 * p1 + p2 * p2

    cb0 = c0.astype(jnp.bfloat16).astype(f32)
    cb1 = c1.astype(jnp.bfloat16).astype(f32)
    cb2 = c2.astype(jnp.bfloat16).astype(f32)
    pb0 = p0.astype(jnp.bfloat16).astype(f32)
    pb1 = p1.astype(jnp.bfloat16).astype(f32)
    pb2 = p2.astype(jnp.bfloat16).astype(f32)

    q0 = cb0 * pb0
    q1 = cb1 * pb1
    q2 = cb2 * pb2
    s = q0 + q1
    bv = s - q0
    av = s - bv
    e = (q0 - av) + (q1 - bv)
    t = s + q2
    bv2 = t - s
    av2 = t - bv2
    f = (s - av2) + (q2 - bv2)
    inner = t + (e + f)

    d2 = jnp.maximum((a_sq + b_sq) - 2.0 * inner, 0.0)
    dist = jnp.sqrt(d2 + 1e-08)

    nchunk = N // NLANE
    INF = jnp.float32(jnp.inf)

    svals = [jnp.full((MT, NLANE), INF, f32) for _ in range(LLIST)]
    schnk = [jnp.zeros((MT, NLANE), jnp.int32) for _ in range(LLIST)]

    for j in range(nchunk):
        cv = dist[:, j * NLANE:(j + 1) * NLANE]
        ci = jnp.full((MT, NLANE), j, jnp.int32)
        for sl in range(LLIST):
            lt = cv < svals[sl]
            nsv = jnp.where(lt, cv, svals[sl])
            nsc = jnp.where(lt, ci, schnk[sl])
            cv = jnp.where(lt, svals[sl], cv)
            ci = jnp.where(lt, schnk[sl], ci)
            svals[sl] = nsv
            schnk[sl] = nsc

    lane = jax.lax.broadcasted_iota(jnp.int32, (MT, NLANE), 1)
    nidx = [schnk[sl] * NLANE + lane for sl in range(LLIST)]

    BIGI = jnp.int32(2**30)
    winners = []
    for _k in range(KNN):
        m = svals[0]
        for sl in range(1, LLIST):
            m = jnp.minimum(m, svals[sl])
        rowmin = jnp.min(m, axis=-1, keepdims=True)
        cand = jnp.full((MT, NLANE), BIGI, jnp.int32)
        for sl in range(LLIST):
            cand = jnp.minimum(cand, jnp.where(svals[sl] == rowmin, nidx[sl], BIGI))
        rowidx = jnp.min(cand, axis=-1, keepdims=True)
        winners.append(rowidx)
        for sl in range(LLIST):
            svals[sl] = jnp.where(nidx[sl] == rowidx, INF, svals[sl])

    o_ref[0] = jnp.concatenate(winners, axis=1)


def _dist_topk(centers, points_t):
    B, M, _ = centers.shape
    N = points_t.shape[2]
    MT = 8
    return pl.pallas_call(
        _dist_topk_kernel,
        grid=(B, M // MT),
        in_specs=[
            pl.BlockSpec((1, MT, 3), lambda b, i: (b, i, 0)),
            pl.BlockSpec((1, 3, N), lambda b, i: (b, 0, 0)),
        ],
        out_specs=pl.BlockSpec((1, MT, KNN), lambda b, i: (b, i, 0)),
        out_shape=jax.ShapeDtypeStruct((B, M, KNN), jnp.int32),
    )(centers, points_t)


def _mlp_kernel(nf_ref, cf_ref, w1a_ref, w1b_ref, w2_ref, w3_ref,
                b1_ref, b2_ref, b3_ref, o_ref):
    MC = cf_ref.shape[0]
    C = cf_ref.shape[1]
    K = KNN
    bf16 = jnp.bfloat16
    f32 = jnp.float32
    nf = nf_ref[...]
    cf = cf_ref[...]
    w1a = w1a_ref[...].astype(bf16)
    w1b = w1b_ref[...].astype(bf16)
    w2 = w2_ref[...].astype(bf16)
    w3 = w3_ref[...].astype(bf16)
    b1 = b1_ref[...]
    b2 = b2_ref[...]
    b3 = b3_ref[...]

    a = jnp.dot(cf.astype(bf16), w1a, preferred_element_type=f32)
    a_big = jnp.broadcast_to(a[:, None, :], (MC, K, C)).reshape(MC * K, C)
    h = jax.nn.relu(a_big + b1
                    + jnp.dot(nf.astype(bf16), w1b, preferred_element_type=f32))
    w = jax.nn.sigmoid(jnp.dot(h.astype(bf16), w2, preferred_element_type=f32)
                       + b2)
    pm = (nf * w).reshape(MC, K, C).sum(axis=1) * (1.0 / K)
    fused = pm + cf
    o_ref[...] = jax.nn.relu(jnp.dot(fused.astype(bf16), w3,
                                     preferred_element_type=f32) + b3)


def _mlp(nf2, cf2, W1aT, W1bT, W2T, W3T, b1, b2, b3):
    T, C = cf2.shape
    MC = 64
    return pl.pallas_call(
        _mlp_kernel,
        grid=(T // MC,),
        in_specs=[
            pl.BlockSpec((MC * KNN, C), lambda i: (i, 0)),
            pl.BlockSpec((MC, C), lambda i: (i, 0)),
            pl.BlockSpec((C, C), lambda i: (0, 0)),
            pl.BlockSpec((C, C), lambda i: (0, 0)),
            pl.BlockSpec((C, C), lambda i: (0, 0)),
            pl.BlockSpec((C, C), lambda i: (0, 0)),
            pl.BlockSpec((1, C), lambda i: (0, 0)),
            pl.BlockSpec((1, C), lambda i: (0, 0)),
            pl.BlockSpec((1, C), lambda i: (0, 0)),
        ],
        out_specs=pl.BlockSpec((MC, C), lambda i: (i, 0)),
        out_shape=jax.ShapeDtypeStruct((T, C), jnp.float32),
    )(nf2, cf2, W1aT, W1bT, W2T, W3T, b1, b2, b3)


def kernel(points, feats, center_idx, W1, b1, W2, b2, W3, b3):
    Bn, Nn, _ = points.shape
    Mn = center_idx.shape[1]
    K = KNN
    C = feats.shape[-1]
    centers = jnp.take_along_axis(
        points, jnp.broadcast_to(center_idx[:, :, None], (Bn, Mn, 3)), axis=1)
    points_t = jnp.transpose(points, (0, 2, 1))
    knn_idx = _dist_topk(centers, points_t)
    neigh_feats = feats[jnp.arange(Bn)[:, None, None], knn_idx]
    centers_feats = jnp.take_along_axis(
        feats, jnp.broadcast_to(center_idx[:, :, None], (Bn, Mn, C)), axis=1)
    nf2 = neigh_feats.reshape(Bn * Mn * K, C)
    cf2 = centers_feats.reshape(Bn * Mn, C)
    out2 = _mlp(nf2, cf2,
                W1[:, :C].T, W1[:, C:].T, W2.T, W3.T,
                b1.reshape(1, C), b2.reshape(1, C), b3.reshape(1, C))
    out = out2.reshape(Bn, Mn, C)
    return (out, knn_idx)


# speedup vs baseline: 3.2341x; 1.8230x over previous
import jax
import jax.numpy as jnp
from jax.experimental import pallas as pl

MAXR = 10.0
KNN = 32
NLANE = 128
LLIST = 8


def _dist_topk_kernel(c_ref, pt_ref, o_ref):
    c = c_ref[0]
    pt = pt_ref[0]
    MT = c.shape[0]
    N = pt.shape[1]
    f32 = jnp.float32

    c0 = c[:, 0:1]
    c1 = c[:, 1:2]
    c2 = c[:, 2:3]
    p0 = pt[0:1, :]
    p1 = pt[1:2, :]
    p2 = pt[2:3, :]

    a_sq = c0 * c0 + c1 * c1 + c2 * c2
    b_sq = p0 * p0 + p1 * p1 + p2 * p2

    cb0 = c0.astype(jnp.bfloat16).astype(f32)
    cb1 = c1.astype(jnp.bfloat16).astype(f32)
    cb2 = c2.astype(jnp.bfloat16).astype(f32)
    pb0 = p0.astype(jnp.bfloat16).astype(f32)
    pb1 = p1.astype(jnp.bfloat16).astype(f32)
    pb2 = p2.astype(jnp.bfloat16).astype(f32)

    q0 = cb0 * pb0
    q1 = cb1 * pb1
    q2 = cb2 * pb2
    s = q0 + q1
    bv = s - q0
    av = s - bv
    e = (q0 - av) + (q1 - bv)
    t = s + q2
    bv2 = t - s
    av2 = t - bv2
    f = (s - av2) + (q2 - bv2)
    inner = t + (e + f)

    d2 = jnp.maximum((a_sq + b_sq) - 2.0 * inner, 0.0)
    dist = jnp.sqrt(d2 + 1e-08)

    nchunk = N // NLANE
    INF = jnp.float32(jnp.inf)

    svals = [jnp.full((MT, NLANE), INF, f32) for _ in range(LLIST)]
    schnk = [jnp.zeros((MT, NLANE), jnp.int32) for _ in range(LLIST)]

    for j in range(nchunk):
        cv = dist[:, j * NLANE:(j + 1) * NLANE]
        ci = jnp.full((MT, NLANE), j, jnp.int32)
        for sl in range(LLIST):
            lt = cv < svals[sl]
            nsv = jnp.where(lt, cv, svals[sl])
            nsc = jnp.where(lt, ci, schnk[sl])
            cv = jnp.where(lt, svals[sl], cv)
            ci = jnp.where(lt, schnk[sl], ci)
            svals[sl] = nsv
            schnk[sl] = nsc

    lane = jax.lax.broadcasted_iota(jnp.int32, (MT, NLANE), 1)
    nidx = [schnk[sl] * NLANE + lane for sl in range(LLIST)]

    BIGI = jnp.int32(2**30)
    winners = []
    for _k in range(KNN):
        m = svals[0]
        for sl in range(1, LLIST):
            m = jnp.minimum(m, svals[sl])
        rowmin = jnp.min(m, axis=-1, keepdims=True)
        cand = jnp.full((MT, NLANE), BIGI, jnp.int32)
        for sl in range(LLIST):
            cand = jnp.minimum(cand, jnp.where(svals[sl] == rowmin, nidx[sl], BIGI))
        rowidx = jnp.min(cand, axis=-1, keepdims=True)
        winners.append(rowidx)
        for sl in range(LLIST):
            svals[sl] = jnp.where(nidx[sl] == rowidx, INF, svals[sl])

    o_ref[0] = jnp.concatenate(winners, axis=1)


def _dist_topk(centers, points_t):
    B, M, _ = centers.shape
    N = points_t.shape[2]
    MT = 32
    return pl.pallas_call(
        _dist_topk_kernel,
        grid=(B, M // MT),
        in_specs=[
            pl.BlockSpec((1, MT, 3), lambda b, i: (b, i, 0)),
            pl.BlockSpec((1, 3, N), lambda b, i: (b, 0, 0)),
        ],
        out_specs=pl.BlockSpec((1, MT, KNN), lambda b, i: (b, i, 0)),
        out_shape=jax.ShapeDtypeStruct((B, M, KNN), jnp.int32),
    )(centers, points_t)


def _mlp_kernel(nf_ref, cf_ref, w1a_ref, w1b_ref, w2_ref, w3_ref,
                b1_ref, b2_ref, b3_ref, o_ref):
    MC = cf_ref.shape[0]
    C = cf_ref.shape[1]
    K = KNN
    bf16 = jnp.bfloat16
    f32 = jnp.float32
    nf = nf_ref[...]
    cf = cf_ref[...]
    w1a = w1a_ref[...].astype(bf16)
    w1b = w1b_ref[...].astype(bf16)
    w2 = w2_ref[...].astype(bf16)
    w3 = w3_ref[...].astype(bf16)
    b1 = b1_ref[...]
    b2 = b2_ref[...]
    b3 = b3_ref[...]

    a = jnp.dot(cf.astype(bf16), w1a, preferred_element_type=f32)
    a_big = jnp.broadcast_to(a[:, None, :], (MC, K, C)).reshape(MC * K, C)
    h = jax.nn.relu(a_big + b1
                    + jnp.dot(nf.astype(bf16), w1b, preferred_element_type=f32))
    w = jax.nn.sigmoid(jnp.dot(h.astype(bf16), w2, preferred_element_type=f32)
                       + b2)
    pm = (nf * w).reshape(MC, K, C).sum(axis=1) * (1.0 / K)
    fused = pm + cf
    o_ref[...] = jax.nn.relu(jnp.dot(fused.astype(bf16), w3,
                                     preferred_element_type=f32) + b3)


def _mlp(nf2, cf2, W1aT, W1bT, W2T, W3T, b1, b2, b3):
    T, C = cf2.shape
    MC = 64
    return pl.pallas_call(
        _mlp_kernel,
        grid=(T // MC,),
        in_specs=[
            pl.BlockSpec((MC * KNN, C), lambda i: (i, 0)),
            pl.BlockSpec((MC, C), lambda i: (i, 0)),
            pl.BlockSpec((C, C), lambda i: (0, 0)),
            pl.BlockSpec((C, C), lambda i: (0, 0)),
            pl.BlockSpec((C, C), lambda i: (0, 0)),
            pl.BlockSpec((C, C), lambda i: (0, 0)),
            pl.BlockSpec((1, C), lambda i: (0, 0)),
            pl.BlockSpec((1, C), lambda i: (0, 0)),
            pl.BlockSpec((1, C), lambda i: (0, 0)),
        ],
        out_specs=pl.BlockSpec((MC, C), lambda i: (i, 0)),
        out_shape=jax.ShapeDtypeStruct((T, C), jnp.float32),
    )(nf2, cf2, W1aT, W1bT, W2T, W3T, b1, b2, b3)


def kernel(points, feats, center_idx, W1, b1, W2, b2, W3, b3):
    Bn, Nn, _ = points.shape
    Mn = center_idx.shape[1]
    K = KNN
    C = feats.shape[-1]
    centers = jnp.take_along_axis(
        points, jnp.broadcast_to(center_idx[:, :, None], (Bn, Mn, 3)), axis=1)
    points_t = jnp.transpose(points, (0, 2, 1))
    knn_idx = _dist_topk(centers, points_t)
    neigh_feats = feats[jnp.arange(Bn)[:, None, None], knn_idx]
    centers_feats = jnp.take_along_axis(
        feats, jnp.broadcast_to(center_idx[:, :, None], (Bn, Mn, C)), axis=1)
    nf2 = neigh_feats.reshape(Bn * Mn * K, C)
    cf2 = centers_feats.reshape(Bn * Mn, C)
    out2 = _mlp(nf2, cf2,
                W1[:, :C].T, W1[:, C:].T, W2.T, W3.T,
                b1.reshape(1, C), b2.reshape(1, C), b3.reshape(1, C))
    out = out2.reshape(Bn, Mn, C)
    return (out, knn_idx)


# speedup vs baseline: 3.7410x; 1.1567x over previous
import jax
import jax.numpy as jnp
from jax.experimental import pallas as pl

MAXR = 10.0
KNN = 32
NLANE = 128
LLIST = 8


def _dist_topk_kernel(c_ref, pt_ref, o_ref):
    c = c_ref[0]
    pt = pt_ref[0]
    MT = c.shape[0]
    N = pt.shape[1]
    f32 = jnp.float32

    c0 = c[:, 0:1]
    c1 = c[:, 1:2]
    c2 = c[:, 2:3]
    p0 = pt[0:1, :]
    p1 = pt[1:2, :]
    p2 = pt[2:3, :]

    a_sq = c0 * c0 + c1 * c1 + c2 * c2
    b_sq = p0 * p0 + p1 * p1 + p2 * p2

    cb0 = c0.astype(jnp.bfloat16).astype(f32)
    cb1 = c1.astype(jnp.bfloat16).astype(f32)
    cb2 = c2.astype(jnp.bfloat16).astype(f32)
    pb0 = p0.astype(jnp.bfloat16).astype(f32)
    pb1 = p1.astype(jnp.bfloat16).astype(f32)
    pb2 = p2.astype(jnp.bfloat16).astype(f32)

    q0 = cb0 * pb0
    q1 = cb1 * pb1
    q2 = cb2 * pb2
    s = q0 + q1
    bv = s - q0
    av = s - bv
    e = (q0 - av) + (q1 - bv)
    t = s + q2
    bv2 = t - s
    av2 = t - bv2
    f = (s - av2) + (q2 - bv2)
    inner = t + (e + f)

    d2 = jnp.maximum((a_sq + b_sq) - 2.0 * inner, 0.0)
    dist = jnp.sqrt(d2 + 1e-08)

    nchunk = N // NLANE
    INF = jnp.float32(jnp.inf)

    svals = [jnp.full((MT, NLANE), INF, f32) for _ in range(LLIST)]
    schnk = [jnp.zeros((MT, NLANE), jnp.int32) for _ in range(LLIST)]

    for j in range(nchunk):
        cv = dist[:, j * NLANE:(j + 1) * NLANE]
        ci = jnp.full((MT, NLANE), j, jnp.int32)
        for sl in range(LLIST):
            lt = cv < svals[sl]
            nsv = jnp.where(lt, cv, svals[sl])
            nsc = jnp.where(lt, ci, schnk[sl])
            cv = jnp.where(lt, svals[sl], cv)
            ci = jnp.where(lt, schnk[sl], ci)
            svals[sl] = nsv
            schnk[sl] = nsc

    lane = jax.lax.broadcasted_iota(jnp.int32, (MT, NLANE), 1)
    nidx = [schnk[sl] * NLANE + lane for sl in range(LLIST)]

    BIGI = jnp.int32(2**30)
    winners = []
    for _k in range(KNN):
        m = svals[0]
        for sl in range(1, LLIST):
            m = jnp.minimum(m, svals[sl])
        rowmin = jnp.min(m, axis=-1, keepdims=True)
        cand = jnp.full((MT, NLANE), BIGI, jnp.int32)
        for sl in range(LLIST):
            cand = jnp.minimum(cand, jnp.where(svals[sl] == rowmin, nidx[sl], BIGI))
        rowidx = jnp.min(cand, axis=-1, keepdims=True)
        winners.append(rowidx)
        for sl in range(LLIST):
            svals[sl] = jnp.where(nidx[sl] == rowidx, INF, svals[sl])

    o_ref[0] = jnp.concatenate(winners, axis=1)


def _dist_topk(centers, points_t):
    B, M, _ = centers.shape
    N = points_t.shape[2]
    MT = 64
    return pl.pallas_call(
        _dist_topk_kernel,
        grid=(B, M // MT),
        in_specs=[
            pl.BlockSpec((1, MT, 3), lambda b, i: (b, i, 0)),
            pl.BlockSpec((1, 3, N), lambda b, i: (b, 0, 0)),
        ],
        out_specs=pl.BlockSpec((1, MT, KNN), lambda b, i: (b, i, 0)),
        out_shape=jax.ShapeDtypeStruct((B, M, KNN), jnp.int32),
    )(centers, points_t)


def _mlp_kernel(nf_ref, cf_ref, w1a_ref, w1b_ref, w2_ref, w3_ref,
                b1_ref, b2_ref, b3_ref, o_ref):
    MC = cf_ref.shape[0]
    C = cf_ref.shape[1]
    K = KNN
    bf16 = jnp.bfloat16
    f32 = jnp.float32
    nf = nf_ref[...]
    cf = cf_ref[...]
    w1a = w1a_ref[...].astype(bf16)
    w1b = w1b_ref[...].astype(bf16)
    w2 = w2_ref[...].astype(bf16)
    w3 = w3_ref[...].astype(bf16)
    b1 = b1_ref[...]
    b2 = b2_ref[...]
    b3 = b3_ref[...]

    a = jnp.dot(cf.astype(bf16), w1a, preferred_element_type=f32)
    a_big = jnp.broadcast_to(a[:, None, :], (MC, K, C)).reshape(MC * K, C)
    h = jax.nn.relu(a_big + b1
                    + jnp.dot(nf.astype(bf16), w1b, preferred_element_type=f32))
    w = jax.nn.sigmoid(jnp.dot(h.astype(bf16), w2, preferred_element_type=f32)
                       + b2)
    pm = (nf * w).reshape(MC, K, C).sum(axis=1) * (1.0 / K)
    fused = pm + cf
    o_ref[...] = jax.nn.relu(jnp.dot(fused.astype(bf16), w3,
                                     preferred_element_type=f32) + b3)


def _mlp(nf2, cf2, W1aT, W1bT, W2T, W3T, b1, b2, b3):
    T, C = cf2.shape
    MC = 64
    return pl.pallas_call(
        _mlp_kernel,
        grid=(T // MC,),
        in_specs=[
            pl.BlockSpec((MC * KNN, C), lambda i: (i, 0)),
            pl.BlockSpec((MC, C), lambda i: (i, 0)),
            pl.BlockSpec((C, C), lambda i: (0, 0)),
            pl.BlockSpec((C, C), lambda i: (0, 0)),
            pl.BlockSpec((C, C), lambda i: (0, 0)),
            pl.BlockSpec((C, C), lambda i: (0, 0)),
            pl.BlockSpec((1, C), lambda i: (0, 0)),
            pl.BlockSpec((1, C), lambda i: (0, 0)),
            pl.BlockSpec((1, C), lambda i: (0, 0)),
        ],
        out_specs=pl.BlockSpec((MC, C), lambda i: (i, 0)),
        out_shape=jax.ShapeDtypeStruct((T, C), jnp.float32),
    )(nf2, cf2, W1aT, W1bT, W2T, W3T, b1, b2, b3)


def kernel(points, feats, center_idx, W1, b1, W2, b2, W3, b3):
    Bn, Nn, _ = points.shape
    Mn = center_idx.shape[1]
    K = KNN
    C = feats.shape[-1]
    centers = jnp.take_along_axis(
        points, jnp.broadcast_to(center_idx[:, :, None], (Bn, Mn, 3)), axis=1)
    points_t = jnp.transpose(points, (0, 2, 1))
    knn_idx = _dist_topk(centers, points_t)
    neigh_feats = feats[jnp.arange(Bn)[:, None, None], knn_idx]
    centers_feats = jnp.take_along_axis(
        feats, jnp.broadcast_to(center_idx[:, :, None], (Bn, Mn, C)), axis=1)
    nf2 = neigh_feats.reshape(Bn * Mn * K, C)
    cf2 = centers_feats.reshape(Bn * Mn, C)
    out2 = _mlp(nf2, cf2,
                W1[:, :C].T, W1[:, C:].T, W2.T, W3.T,
                b1.reshape(1, C), b2.reshape(1, C), b3.reshape(1, C))
    out = out2.reshape(Bn, Mn, C)
    return (out, knn_idx)
